# baseline (speedup 1.0000x reference)
import functools

import numpy as np
import jax
import jax.numpy as jnp
from jax import lax
from jax.experimental import pallas as pl
from jax.experimental.pallas import tpu as pltpu
from jax.experimental.pallas import tpu_sc as plsc

_N = 10000
_E = 320000
_DF = 128
_D = 64
_NLAYERS = 8
_ALPHA = 0.1
_LAMDA = 0.5

_NP = 10240
_ROWS_PER_TILE = _NP // 16
_CHUNK = 256
_NWORK = 32
_CPW = 40
_NPAIR = _CPW // 2
_EP = _CHUNK * _CPW * _NWORK

_mesh = plsc.VectorSubcoreMesh(core_axis_name="c", subcore_axis_name="s")


@functools.partial(
    pl.kernel,
    out_type=jax.ShapeDtypeStruct((2, _NP, _D), jnp.float32),
    mesh=_mesh,
    compiler_params=pltpu.CompilerParams(use_tc_tiling_on_sc=False),
    scratch_types=[
        pltpu.VMEM((_CHUNK,), jnp.int32),
        pltpu.VMEM((_CHUNK,), jnp.int32),
        pltpu.VMEM((_CHUNK,), jnp.int32),
        pltpu.VMEM((_CHUNK,), jnp.int32),
        pltpu.VMEM((_CHUNK, _D), jnp.float32),
        pltpu.VMEM((_CHUNK, _D), jnp.float32),
        pltpu.VMEM_SHARED((_NP, _D), jnp.float32),
        pltpu.VMEM_SHARED((_NP, _D), jnp.float32),
        pltpu.SemaphoreType.DMA,
        pltpu.SemaphoreType.DMA,
        pltpu.SemaphoreType.DMA,
        pltpu.SemaphoreType.DMA,
    ],
)
def _sc_agg(hs, srcp, dstp, zeros, out, idx_sa, idx_da, idx_sb, idx_db,
            rows_a, rows_b, table, acc, gsa, gsb, ssa, ssb):
    cid = lax.axis_index("c")
    sid = lax.axis_index("s")
    wid = cid * 16 + sid
    r0 = sid * _ROWS_PER_TILE
    pltpu.sync_copy(hs.at[pl.ds(r0, _ROWS_PER_TILE)],
                    table.at[pl.ds(r0, _ROWS_PER_TILE)])
    pltpu.sync_copy(zeros.at[pl.ds(r0, _ROWS_PER_TILE)],
                    acc.at[pl.ds(r0, _ROWS_PER_TILE)])
    plsc.subcore_barrier()
    base = wid * (_CPW * _CHUNK)

    def body(j, carry):
        c0 = base + j * (2 * _CHUNK)
        c1 = c0 + _CHUNK
        pltpu.sync_copy(srcp.at[pl.ds(c0, _CHUNK)], idx_sa)
        pltpu.sync_copy(dstp.at[pl.ds(c0, _CHUNK)], idx_da)
        ga = pltpu.async_copy(table.at[idx_sa], rows_a, gsa)
        pltpu.sync_copy(srcp.at[pl.ds(c1, _CHUNK)], idx_sb)
        pltpu.sync_copy(dstp.at[pl.ds(c1, _CHUNK)], idx_db)
        gb = pltpu.async_copy(table.at[idx_sb], rows_b, gsb)
        ga.wait()
        sa = pltpu.async_copy(rows_a, acc.at[idx_da], ssa, add=True)
        gb.wait()
        sb = pltpu.async_copy(rows_b, acc.at[idx_db], ssb, add=True)
        sa.wait()
        sb.wait()
        return carry

    lax.fori_loop(0, _NPAIR, body, 0)
    plsc.subcore_barrier()
    pltpu.sync_copy(acc.at[pl.ds(r0, _ROWS_PER_TILE)],
                    out.at[cid, pl.ds(r0, _ROWS_PER_TILE)])


_BLK = 1024
_GRID = _NP // _BLK


def _tc_h0_body(x_ref, w0_ref, b0_ref, p0_ref, p1_ref,
                h0_ref, hs_ref, dinv_ref):
    deg = p0_ref[:, 0:1] + p1_ref[:, 0:1]
    dinv = lax.rsqrt(jnp.maximum(deg, 1.0))
    h0 = jnp.maximum(
        jnp.dot(x_ref[...], w0_ref[...], preferred_element_type=jnp.float32)
        + b0_ref[...], 0.0)
    h0_ref[...] = h0
    hs_ref[...] = h0 * dinv
    dinv_ref[...] = dinv


def _tc_h0(xp, w0, b0, p0, p1):
    return pl.pallas_call(
        _tc_h0_body,
        grid=(_GRID,),
        in_specs=[
            pl.BlockSpec((_BLK, _DF), lambda i: (i, 0)),
            pl.BlockSpec((_DF, _D), lambda i: (0, 0)),
            pl.BlockSpec((1, _D), lambda i: (0, 0)),
            pl.BlockSpec((_BLK, _D), lambda i: (i, 0)),
            pl.BlockSpec((_BLK, _D), lambda i: (i, 0)),
        ],
        out_specs=[
            pl.BlockSpec((_BLK, _D), lambda i: (i, 0)),
            pl.BlockSpec((_BLK, _D), lambda i: (i, 0)),
            pl.BlockSpec((_BLK, 1), lambda i: (i, 0)),
        ],
        out_shape=[
            jax.ShapeDtypeStruct((_NP, _D), jnp.float32),
            jax.ShapeDtypeStruct((_NP, _D), jnp.float32),
            jax.ShapeDtypeStruct((_NP, 1), jnp.float32),
        ],
    )(xp, w0, b0, p0, p1)


def _tc_layer_body(beta, p0_ref, p1_ref, dinv_ref, h0_ref, w_ref, hs_ref):
    agg = (p0_ref[...] + p1_ref[...]) * dinv_ref[...]
    support = (1.0 - _ALPHA) * agg + _ALPHA * h0_ref[...]
    t = (1.0 - beta) * support + beta * jnp.dot(
        support, w_ref[...], preferred_element_type=jnp.float32)
    hs_ref[...] = jnp.maximum(t, 0.0) * dinv_ref[...]


def _tc_layer(beta, p0, p1, dinv, h0, w):
    return pl.pallas_call(
        functools.partial(_tc_layer_body, beta),
        grid=(_GRID,),
        in_specs=[
            pl.BlockSpec((_BLK, _D), lambda i: (i, 0)),
            pl.BlockSpec((_BLK, _D), lambda i: (i, 0)),
            pl.BlockSpec((_BLK, 1), lambda i: (i, 0)),
            pl.BlockSpec((_BLK, _D), lambda i: (i, 0)),
            pl.BlockSpec((_D, _D), lambda i: (0, 0)),
        ],
        out_specs=pl.BlockSpec((_BLK, _D), lambda i: (i, 0)),
        out_shape=jax.ShapeDtypeStruct((_NP, _D), jnp.float32),
    )(p0, p1, dinv, h0, w)


def _tc_final_body(beta, p0_ref, p1_ref, dinv_ref, h0_ref, w_ref,
                   wout_ref, bout_ref, out_ref):
    agg = (p0_ref[...] + p1_ref[...]) * dinv_ref[...]
    support = (1.0 - _ALPHA) * agg + _ALPHA * h0_ref[...]
    t = (1.0 - beta) * support + beta * jnp.dot(
        support, w_ref[...], preferred_element_type=jnp.float32)
    h = jnp.maximum(t, 0.0)
    out_ref[...] = jnp.dot(
        h, wout_ref[...], preferred_element_type=jnp.float32) + bout_ref[...]


def _tc_final(beta, p0, p1, dinv, h0, w, wout, bout):
    return pl.pallas_call(
        functools.partial(_tc_final_body, beta),
        grid=(_GRID,),
        in_specs=[
            pl.BlockSpec((_BLK, _D), lambda i: (i, 0)),
            pl.BlockSpec((_BLK, _D), lambda i: (i, 0)),
            pl.BlockSpec((_BLK, 1), lambda i: (i, 0)),
            pl.BlockSpec((_BLK, _D), lambda i: (i, 0)),
            pl.BlockSpec((_D, _D), lambda i: (0, 0)),
            pl.BlockSpec((_D, _D), lambda i: (0, 0)),
            pl.BlockSpec((1, _D), lambda i: (0, 0)),
        ],
        out_specs=pl.BlockSpec((_BLK, _D), lambda i: (i, 0)),
        out_shape=jax.ShapeDtypeStruct((_NP, _D), jnp.float32),
    )(p0, p1, dinv, h0, w, wout, bout)


def kernel(features, edge_index, W0, b0, Ws, W_out, b_out):
    src = edge_index[0]
    dst = edge_index[1]
    pad = jnp.full((_EP - _E,), _N, jnp.int32)
    srcp = jnp.concatenate([src, pad])
    dstp = jnp.concatenate([dst, pad])
    zeros = jnp.zeros((_NP, _D), jnp.float32)
    ones = jnp.ones((_NP, _D), jnp.float32)
    xp = jnp.concatenate(
        [features, jnp.zeros((_NP - _N, _DF), jnp.float32)], axis=0)

    degp = _sc_agg(ones, srcp, dstp, zeros)
    h0, hs, dinv = _tc_h0(xp, W0, b0.reshape(1, _D), degp[0], degp[1])
    out = None
    for i in range(_NLAYERS):
        beta = float(np.log(_LAMDA / (i + 1) + 1.0))
        p = _sc_agg(hs, srcp, dstp, zeros)
        if i < _NLAYERS - 1:
            hs = _tc_layer(beta, p[0], p[1], dinv, h0, Ws[i])
        else:
            out = _tc_final(beta, p[0], p[1], dinv, h0, Ws[i],
                            W_out, b_out.reshape(1, _D))
    return out[:_N]


# speedup vs baseline: 1.1321x; 1.1321x over previous
import functools

import numpy as np
import jax
import jax.numpy as jnp
from jax import lax
from jax.experimental import pallas as pl
from jax.experimental.pallas import tpu as pltpu
from jax.experimental.pallas import tpu_sc as plsc

_N = 10000
_E = 320000
_DF = 128
_D = 64
_NLAYERS = 8
_ALPHA = 0.1
_LAMDA = 0.5

_NP = 10240
_ROWS_PER_TILE = _NP // 16
_CHUNK = 512
_NWORK = 32
_CPW = 20
_EP = _CHUNK * _CPW * _NWORK

_mesh = plsc.VectorSubcoreMesh(core_axis_name="c", subcore_axis_name="s")


@functools.partial(
    pl.kernel,
    out_type=jax.ShapeDtypeStruct((2, _NP, _D), jnp.float32),
    mesh=_mesh,
    compiler_params=pltpu.CompilerParams(use_tc_tiling_on_sc=False),
    scratch_types=[
        pltpu.VMEM((_CHUNK,), jnp.int32),
        pltpu.VMEM((_CHUNK,), jnp.int32),
        pltpu.VMEM((_CHUNK, _D), jnp.float32),
        pltpu.VMEM_SHARED((_NP, _D), jnp.float32),
        pltpu.VMEM_SHARED((_NP, _D), jnp.float32),
        pltpu.SemaphoreType.DMA,
    ],
)
def _sc_agg(hs, srcp, dstp, zeros, out, idx_s, idx_d, rows, table, acc, sem):
    cid = lax.axis_index("c")
    sid = lax.axis_index("s")
    wid = cid * 16 + sid
    r0 = sid * _ROWS_PER_TILE
    pltpu.sync_copy(hs.at[pl.ds(r0, _ROWS_PER_TILE)],
                    table.at[pl.ds(r0, _ROWS_PER_TILE)])
    pltpu.sync_copy(zeros.at[pl.ds(r0, _ROWS_PER_TILE)],
                    acc.at[pl.ds(r0, _ROWS_PER_TILE)])
    plsc.subcore_barrier()
    base = wid * (_CPW * _CHUNK)

    def body(j, carry):
        off = base + j * _CHUNK
        pltpu.sync_copy(srcp.at[pl.ds(off, _CHUNK)], idx_s)
        pltpu.sync_copy(dstp.at[pl.ds(off, _CHUNK)], idx_d)
        pltpu.async_copy(table.at[idx_s], rows, sem).wait()
        pltpu.sync_copy(rows, acc.at[idx_d], add=True)
        return carry

    lax.fori_loop(0, _CPW, body, 0)
    plsc.subcore_barrier()
    pltpu.sync_copy(acc.at[pl.ds(r0, _ROWS_PER_TILE)],
                    out.at[cid, pl.ds(r0, _ROWS_PER_TILE)])


_DDEG = 8


@functools.partial(
    pl.kernel,
    out_type=jax.ShapeDtypeStruct((2, _NP, _DDEG), jnp.float32),
    mesh=_mesh,
    compiler_params=pltpu.CompilerParams(use_tc_tiling_on_sc=False),
    scratch_types=[
        pltpu.VMEM((_CHUNK,), jnp.int32),
        pltpu.VMEM((_CHUNK, _DDEG), jnp.float32),
        pltpu.VMEM_SHARED((_NP, _DDEG), jnp.float32),
    ],
)
def _sc_deg(ones, dstp, zeros, out, idx_d, rows, acc):
    cid = lax.axis_index("c")
    sid = lax.axis_index("s")
    wid = cid * 16 + sid
    r0 = sid * _ROWS_PER_TILE
    pltpu.sync_copy(ones, rows)
    pltpu.sync_copy(zeros.at[pl.ds(r0, _ROWS_PER_TILE)],
                    acc.at[pl.ds(r0, _ROWS_PER_TILE)])
    plsc.subcore_barrier()
    base = wid * (_CPW * _CHUNK)

    def body(j, carry):
        off = base + j * _CHUNK
        pltpu.sync_copy(dstp.at[pl.ds(off, _CHUNK)], idx_d)
        pltpu.sync_copy(rows, acc.at[idx_d], add=True)
        return carry

    lax.fori_loop(0, _CPW, body, 0)
    plsc.subcore_barrier()
    pltpu.sync_copy(acc.at[pl.ds(r0, _ROWS_PER_TILE)],
                    out.at[cid, pl.ds(r0, _ROWS_PER_TILE)])


_BLK = 1024
_GRID = _NP // _BLK


def _tc_h0_body(x_ref, w0_ref, b0_ref, dp_ref,
                h0_ref, hs_ref, dinv_ref):
    deg = dp_ref[0, :, 0:1] + dp_ref[1, :, 0:1]
    dinv = lax.rsqrt(jnp.maximum(deg, 1.0))
    h0 = jnp.maximum(
        jnp.dot(x_ref[...], w0_ref[...], preferred_element_type=jnp.float32)
        + b0_ref[...], 0.0)
    h0_ref[...] = h0
    hs_ref[...] = h0 * dinv
    dinv_ref[...] = dinv


def _tc_h0(xp, w0, b0, dp):
    return pl.pallas_call(
        _tc_h0_body,
        grid=(_GRID,),
        in_specs=[
            pl.BlockSpec((_BLK, _DF), lambda i: (i, 0)),
            pl.BlockSpec((_DF, _D), lambda i: (0, 0)),
            pl.BlockSpec((1, _D), lambda i: (0, 0)),
            pl.BlockSpec((2, _BLK, _DDEG), lambda i: (0, i, 0)),
        ],
        out_specs=[
            pl.BlockSpec((_BLK, _D), lambda i: (i, 0)),
            pl.BlockSpec((_BLK, _D), lambda i: (i, 0)),
            pl.BlockSpec((_BLK, 1), lambda i: (i, 0)),
        ],
        out_shape=[
            jax.ShapeDtypeStruct((_NP, _D), jnp.float32),
            jax.ShapeDtypeStruct((_NP, _D), jnp.float32),
            jax.ShapeDtypeStruct((_NP, 1), jnp.float32),
        ],
    )(xp, w0, b0, dp)


def _tc_layer_body(beta, p_ref, dinv_ref, h0_ref, w_ref, hs_ref):
    agg = (p_ref[0] + p_ref[1]) * dinv_ref[...]
    support = (1.0 - _ALPHA) * agg + _ALPHA * h0_ref[...]
    t = (1.0 - beta) * support + beta * jnp.dot(
        support, w_ref[...], preferred_element_type=jnp.float32)
    hs_ref[...] = jnp.maximum(t, 0.0) * dinv_ref[...]


def _tc_layer(beta, p, dinv, h0, w):
    return pl.pallas_call(
        functools.partial(_tc_layer_body, beta),
        grid=(_GRID,),
        in_specs=[
            pl.BlockSpec((2, _BLK, _D), lambda i: (0, i, 0)),
            pl.BlockSpec((_BLK, 1), lambda i: (i, 0)),
            pl.BlockSpec((_BLK, _D), lambda i: (i, 0)),
            pl.BlockSpec((_D, _D), lambda i: (0, 0)),
        ],
        out_specs=pl.BlockSpec((_BLK, _D), lambda i: (i, 0)),
        out_shape=jax.ShapeDtypeStruct((_NP, _D), jnp.float32),
    )(p, dinv, h0, w)


def _tc_final_body(beta, p_ref, dinv_ref, h0_ref, w_ref,
                   wout_ref, bout_ref, out_ref):
    agg = (p_ref[0] + p_ref[1]) * dinv_ref[...]
    support = (1.0 - _ALPHA) * agg + _ALPHA * h0_ref[...]
    t = (1.0 - beta) * support + beta * jnp.dot(
        support, w_ref[...], preferred_element_type=jnp.float32)
    h = jnp.maximum(t, 0.0)
    out_ref[...] = jnp.dot(
        h, wout_ref[...], preferred_element_type=jnp.float32) + bout_ref[...]


def _tc_final(beta, p, dinv, h0, w, wout, bout):
    return pl.pallas_call(
        functools.partial(_tc_final_body, beta),
        grid=(_GRID,),
        in_specs=[
            pl.BlockSpec((2, _BLK, _D), lambda i: (0, i, 0)),
            pl.BlockSpec((_BLK, 1), lambda i: (i, 0)),
            pl.BlockSpec((_BLK, _D), lambda i: (i, 0)),
            pl.BlockSpec((_D, _D), lambda i: (0, 0)),
            pl.BlockSpec((_D, _D), lambda i: (0, 0)),
            pl.BlockSpec((1, _D), lambda i: (0, 0)),
        ],
        out_specs=pl.BlockSpec((_BLK, _D), lambda i: (i, 0)),
        out_shape=jax.ShapeDtypeStruct((_NP, _D), jnp.float32),
    )(p, dinv, h0, w, wout, bout)


def kernel(features, edge_index, W0, b0, Ws, W_out, b_out):
    src = edge_index[0]
    dst = edge_index[1]
    pad = jnp.full((_EP - _E,), _N, jnp.int32)
    srcp = jnp.concatenate([src, pad])
    dstp = jnp.concatenate([dst, pad])
    zeros = jnp.zeros((_NP, _D), jnp.float32)
    zeros_deg = jnp.zeros((_NP, _DDEG), jnp.float32)
    ones_deg = jnp.ones((_CHUNK, _DDEG), jnp.float32)
    xp = jnp.concatenate(
        [features, jnp.zeros((_NP - _N, _DF), jnp.float32)], axis=0)

    degp = _sc_deg(ones_deg, dstp, zeros_deg)
    h0, hs, dinv = _tc_h0(xp, W0, b0.reshape(1, _D), degp)
    out = None
    for i in range(_NLAYERS):
        beta = float(np.log(_LAMDA / (i + 1) + 1.0))
        p = _sc_agg(hs, srcp, dstp, zeros)
        if i < _NLAYERS - 1:
            hs = _tc_layer(beta, p, dinv, h0, Ws[i])
        else:
            out = _tc_final(beta, p, dinv, h0, Ws[i],
                            W_out, b_out.reshape(1, _D))
    return out[:_N]


# speedup vs baseline: 1.1540x; 1.0194x over previous
import functools

import numpy as np
import jax
import jax.numpy as jnp
from jax import lax
from jax.experimental import pallas as pl
from jax.experimental.pallas import tpu as pltpu
from jax.experimental.pallas import tpu_sc as plsc

_N = 10000
_E = 320000
_DF = 128
_D = 64
_NLAYERS = 8
_ALPHA = 0.1
_LAMDA = 0.5

_NP = 10240
_ROWS_PER_TILE = _NP // 16
_CHUNK = 640
_NWORK = 32
_CPW = 16
_EP = _CHUNK * _CPW * _NWORK

_mesh = plsc.VectorSubcoreMesh(core_axis_name="c", subcore_axis_name="s")


@functools.partial(
    pl.kernel,
    out_type=jax.ShapeDtypeStruct((2, _NP, _D), jnp.float32),
    mesh=_mesh,
    compiler_params=pltpu.CompilerParams(use_tc_tiling_on_sc=False),
    scratch_types=[
        pltpu.VMEM((_CHUNK,), jnp.int32),
        pltpu.VMEM((_CHUNK,), jnp.int32),
        pltpu.VMEM((_CHUNK, _D), jnp.float32),
        pltpu.VMEM_SHARED((_NP, _D), jnp.float32),
        pltpu.VMEM_SHARED((_NP, _D), jnp.float32),
        pltpu.SemaphoreType.DMA,
        pltpu.SemaphoreType.DMA,
    ],
)
def _sc_agg(hs, srcp, dstp, zeros, out, idx_s, idx_d, rows, table, acc,
            sem, sem2):
    cid = lax.axis_index("c")
    sid = lax.axis_index("s")
    wid = cid * 16 + sid
    r0 = sid * _ROWS_PER_TILE
    st = pltpu.async_copy(hs.at[pl.ds(r0, _ROWS_PER_TILE)],
                          table.at[pl.ds(r0, _ROWS_PER_TILE)], sem)
    sz = pltpu.async_copy(zeros.at[pl.ds(r0, _ROWS_PER_TILE)],
                          acc.at[pl.ds(r0, _ROWS_PER_TILE)], sem2)
    st.wait()
    sz.wait()
    plsc.subcore_barrier()
    base = wid * (_CPW * _CHUNK)

    def body(j, carry):
        off = base + j * _CHUNK
        pltpu.sync_copy(srcp.at[pl.ds(off, _CHUNK)], idx_s)
        pltpu.sync_copy(dstp.at[pl.ds(off, _CHUNK)], idx_d)
        pltpu.async_copy(table.at[idx_s], rows, sem).wait()
        pltpu.sync_copy(rows, acc.at[idx_d], add=True)
        return carry

    lax.fori_loop(0, _CPW, body, 0)
    plsc.subcore_barrier()
    pltpu.sync_copy(acc.at[pl.ds(r0, _ROWS_PER_TILE)],
                    out.at[cid, pl.ds(r0, _ROWS_PER_TILE)])


_DDEG = 8


@functools.partial(
    pl.kernel,
    out_type=jax.ShapeDtypeStruct((2, _NP, _DDEG), jnp.float32),
    mesh=_mesh,
    compiler_params=pltpu.CompilerParams(use_tc_tiling_on_sc=False),
    scratch_types=[
        pltpu.VMEM((_CHUNK,), jnp.int32),
        pltpu.VMEM((_CHUNK, _DDEG), jnp.float32),
        pltpu.VMEM_SHARED((_NP, _DDEG), jnp.float32),
    ],
)
def _sc_deg(ones, dstp, zeros, out, idx_d, rows, acc):
    cid = lax.axis_index("c")
    sid = lax.axis_index("s")
    wid = cid * 16 + sid
    r0 = sid * _ROWS_PER_TILE
    pltpu.sync_copy(ones, rows)
    pltpu.sync_copy(zeros.at[pl.ds(r0, _ROWS_PER_TILE)],
                    acc.at[pl.ds(r0, _ROWS_PER_TILE)])
    plsc.subcore_barrier()
    base = wid * (_CPW * _CHUNK)

    def body(j, carry):
        off = base + j * _CHUNK
        pltpu.sync_copy(dstp.at[pl.ds(off, _CHUNK)], idx_d)
        pltpu.sync_copy(rows, acc.at[idx_d], add=True)
        return carry

    lax.fori_loop(0, _CPW, body, 0)
    plsc.subcore_barrier()
    pltpu.sync_copy(acc.at[pl.ds(r0, _ROWS_PER_TILE)],
                    out.at[cid, pl.ds(r0, _ROWS_PER_TILE)])


_BLK = 1024
_GRID = _NP // _BLK


def _tc_proj_body(x_ref, w0_ref, b0_ref, h0_ref):
    h0_ref[...] = jnp.maximum(
        jnp.dot(x_ref[...], w0_ref[...], preferred_element_type=jnp.float32)
        + b0_ref[...], 0.0)


def _tc_proj(xp, w0, b0):
    return pl.pallas_call(
        _tc_proj_body,
        grid=(_GRID,),
        in_specs=[
            pl.BlockSpec((_BLK, _DF), lambda i: (i, 0)),
            pl.BlockSpec((_DF, _D), lambda i: (0, 0)),
            pl.BlockSpec((1, _D), lambda i: (0, 0)),
        ],
        out_specs=pl.BlockSpec((_BLK, _D), lambda i: (i, 0)),
        out_shape=jax.ShapeDtypeStruct((_NP, _D), jnp.float32),
    )(xp, w0, b0)


def _tc_scale_body(h0_ref, dp_ref, hs_ref, dinv_ref):
    deg = dp_ref[0, :, 0:1] + dp_ref[1, :, 0:1]
    dinv = lax.rsqrt(jnp.maximum(deg, 1.0))
    hs_ref[...] = h0_ref[...] * dinv
    dinv_ref[...] = dinv


def _tc_scale(h0, dp):
    return pl.pallas_call(
        _tc_scale_body,
        grid=(_GRID,),
        in_specs=[
            pl.BlockSpec((_BLK, _D), lambda i: (i, 0)),
            pl.BlockSpec((2, _BLK, _DDEG), lambda i: (0, i, 0)),
        ],
        out_specs=[
            pl.BlockSpec((_BLK, _D), lambda i: (i, 0)),
            pl.BlockSpec((_BLK, 1), lambda i: (i, 0)),
        ],
        out_shape=[
            jax.ShapeDtypeStruct((_NP, _D), jnp.float32),
            jax.ShapeDtypeStruct((_NP, 1), jnp.float32),
        ],
    )(h0, dp)


def _tc_layer_body(beta, p_ref, dinv_ref, h0_ref, w_ref, hs_ref):
    agg = (p_ref[0] + p_ref[1]) * dinv_ref[...]
    support = (1.0 - _ALPHA) * agg + _ALPHA * h0_ref[...]
    t = (1.0 - beta) * support + beta * jnp.dot(
        support, w_ref[...], preferred_element_type=jnp.float32)
    hs_ref[...] = jnp.maximum(t, 0.0) * dinv_ref[...]


def _tc_layer(beta, p, dinv, h0, w):
    return pl.pallas_call(
        functools.partial(_tc_layer_body, beta),
        grid=(_GRID,),
        in_specs=[
            pl.BlockSpec((2, _BLK, _D), lambda i: (0, i, 0)),
            pl.BlockSpec((_BLK, 1), lambda i: (i, 0)),
            pl.BlockSpec((_BLK, _D), lambda i: (i, 0)),
            pl.BlockSpec((_D, _D), lambda i: (0, 0)),
        ],
        out_specs=pl.BlockSpec((_BLK, _D), lambda i: (i, 0)),
        out_shape=jax.ShapeDtypeStruct((_NP, _D), jnp.float32),
    )(p, dinv, h0, w)


def _tc_final_body(beta, p_ref, dinv_ref, h0_ref, w_ref,
                   wout_ref, bout_ref, out_ref):
    agg = (p_ref[0] + p_ref[1]) * dinv_ref[...]
    support = (1.0 - _ALPHA) * agg + _ALPHA * h0_ref[...]
    t = (1.0 - beta) * support + beta * jnp.dot(
        support, w_ref[...], preferred_element_type=jnp.float32)
    h = jnp.maximum(t, 0.0)
    out_ref[...] = jnp.dot(
        h, wout_ref[...], preferred_element_type=jnp.float32) + bout_ref[...]


def _tc_final(beta, p, dinv, h0, w, wout, bout):
    return pl.pallas_call(
        functools.partial(_tc_final_body, beta),
        grid=(_GRID,),
        in_specs=[
            pl.BlockSpec((2, _BLK, _D), lambda i: (0, i, 0)),
            pl.BlockSpec((_BLK, 1), lambda i: (i, 0)),
            pl.BlockSpec((_BLK, _D), lambda i: (i, 0)),
            pl.BlockSpec((_D, _D), lambda i: (0, 0)),
            pl.BlockSpec((_D, _D), lambda i: (0, 0)),
            pl.BlockSpec((1, _D), lambda i: (0, 0)),
        ],
        out_specs=pl.BlockSpec((_BLK, _D), lambda i: (i, 0)),
        out_shape=jax.ShapeDtypeStruct((_NP, _D), jnp.float32),
    )(p, dinv, h0, w, wout, bout)


def kernel(features, edge_index, W0, b0, Ws, W_out, b_out):
    src = edge_index[0]
    dst = edge_index[1]
    pad = jnp.full((_EP - _E,), _N, jnp.int32)
    srcp = jnp.concatenate([src, pad])
    dstp = jnp.concatenate([dst, pad])
    zeros = jnp.zeros((_NP, _D), jnp.float32)
    zeros_deg = jnp.zeros((_NP, _DDEG), jnp.float32)
    ones_deg = jnp.ones((_CHUNK, _DDEG), jnp.float32)
    xp = jnp.concatenate(
        [features, jnp.zeros((_NP - _N, _DF), jnp.float32)], axis=0)

    h0 = _tc_proj(xp, W0, b0.reshape(1, _D))
    degp = _sc_deg(ones_deg, dstp, zeros_deg)
    hs, dinv = _tc_scale(h0, degp)
    out = None
    for i in range(_NLAYERS):
        beta = float(np.log(_LAMDA / (i + 1) + 1.0))
        p = _sc_agg(hs, srcp, dstp, zeros)
        if i < _NLAYERS - 1:
            hs = _tc_layer(beta, p, dinv, h0, Ws[i])
        else:
            out = _tc_final(beta, p, dinv, h0, Ws[i],
                            W_out, b_out.reshape(1, _D))
    return out[:_N]
